# baseline (speedup 1.0000x reference)
import jax
import jax.numpy as jnp
from jax.experimental import pallas as pl
from jax.experimental.pallas import tpu as pltpu

_B, _H, _W = 16, 512, 512
_N = _B * _H * _W
_BB = 4
_S = _B // _BB


def _fused_body(x_hbm, m_ref, c_ref, bufx, emask, s_ref, in_sems):
    s = pl.program_id(0)

    @pl.when(s == 0)
    def _prologue():
        s_ref[0, 0] = jnp.float32(0.0)
        for b in range(_S):
            pltpu.make_async_copy(
                x_hbm.at[pl.ds(b * _BB, _BB)],
                bufx.at[pl.ds(b * _BB, _BB)],
                in_sems.at[b],
            ).start()

    @pl.when(s < _S)
    def _phase1():
        pltpu.make_async_copy(
            x_hbm.at[pl.ds(s * _BB, _BB)],
            bufx.at[pl.ds(s * _BB, _BB)],
            in_sems.at[s],
        ).wait()
        x = bufx[pl.ds(s * _BB, _BB)]
        ninf = jnp.float32(-jnp.inf)
        col = jax.lax.broadcasted_iota(jnp.int32, (_BB, _H, _W), 2)
        row = jax.lax.broadcasted_iota(jnp.int32, (_BB, _H, _W), 1)
        m = jnp.maximum(
            jnp.maximum(
                jnp.where(col > 0, pltpu.roll(x, 1, 2), ninf),
                jnp.where(col < _W - 1, pltpu.roll(x, _W - 1, 2), ninf),
            ),
            x,
        )
        pooled = jnp.maximum(
            jnp.maximum(
                jnp.where(row > 0, pltpu.roll(m, 1, 1), ninf),
                jnp.where(row < _H - 1, pltpu.roll(m, _H - 1, 1), ninf),
            ),
            m,
        )
        emask[pl.ds(s * _BB, _BB)] = x == pooled
        s_ref[0, 0] += jnp.sum(x)

    @pl.when(s >= _S)
    def _phase2():
        i = s - _S
        mean = s_ref[0, 0] * jnp.float32(1.0 / _N)
        x = bufx[pl.ds(i * _BB, _BB)]
        e = emask[pl.ds(i * _BB, _BB)]
        mask = e & (x > mean)
        m_ref[...] = mask

        @pl.when(s == _S)
        def _init_cnt():
            c_ref[0, 0] = jnp.int32(0)

        c_ref[0, 0] += jnp.sum(mask.astype(jnp.int32))


def kernel(input):
    x3 = input.reshape(_B, _H, _W)
    mask, cnt = pl.pallas_call(
        _fused_body,
        grid=(2 * _S,),
        in_specs=[pl.BlockSpec(memory_space=pl.ANY)],
        out_specs=[
            pl.BlockSpec((_BB, _H, _W), lambda s: (jnp.maximum(s - _S, 0), 0, 0)),
            pl.BlockSpec(memory_space=pltpu.SMEM),
        ],
        out_shape=[
            jax.ShapeDtypeStruct((_B, _H, _W), jnp.bool_),
            jax.ShapeDtypeStruct((1, 1), jnp.int32),
        ],
        scratch_shapes=[
            pltpu.VMEM((_B, _H, _W), jnp.float32),
            pltpu.VMEM((_B, _H, _W), jnp.bool_),
            pltpu.SMEM((1, 1), jnp.float32),
            pltpu.SemaphoreType.DMA((_S,)),
        ],
    )(x3)
    return mask.reshape(_B, 1, _H, _W), cnt[0, 0]
